# baseline (speedup 1.0000x reference)
import jax
import jax.numpy as jnp
from jax.experimental import pallas as pl
from jax.experimental.pallas import tpu as pltpu


def _ceil_to(x, m):
    return ((x + m - 1) // m) * m


def _enc_conv1_kernel(cols_ref, wce_ref, bce_ref, w1f_ref, b1_ref, o_ref,
                      feat_s):
    t = pl.program_id(1)
    hp2, wp2, cf = feat_s.shape
    h, w = hp2 - 2, wp2 - 2
    hd = o_ref.shape[2]
    th = o_ref.shape[1] // w

    @pl.when(t == 0)
    def _():
        f = jnp.dot(cols_ref[0], wce_ref[...],
                    preferred_element_type=jnp.float32) + bce_ref[...]
        feat_s[...] = jnp.zeros((hp2, wp2, cf), feat_s.dtype)
        feat_s[1:h + 1, 1:w + 1, :] = f.reshape(h, w, cf).astype(feat_s.dtype)

    r0 = t * th
    acc = jnp.zeros((th * w, hd), jnp.float32) + b1_ref[...]
    for kh in range(3):
        for kw in range(3):
            tap = kh * 3 + kw
            sl = feat_s[pl.ds(r0 + kh, th), kw:kw + w, :]
            acc = acc + jnp.dot(sl.reshape(th * w, cf),
                                w1f_ref[tap * cf:(tap + 1) * cf, :],
                                preferred_element_type=jnp.float32)
    o_ref[0] = acc.astype(o_ref.dtype)


def _query_mlp_kernel(x_ref, r_ref, w1r_ref, w2_ref, b2_ref, w3_ref, b3_ref,
                      o_ref):
    four, tm, _ = r_ref.shape
    hd = x_ref.shape[2] // 2
    xs = []
    for s in range(4):
        strip = x_ref[s // 2]
        oc = r_ref[s, :, 5:6] > 0.5
        xs.append(jnp.where(oc, strip[:, hd:], strip[:, :hd]))
    x = jnp.concatenate(xs, axis=0)
    rr = r_ref[...].reshape(four * tm, r_ref.shape[2])
    h = x.astype(jnp.float32) + jnp.dot(rr, w1r_ref[...],
                                        preferred_element_type=jnp.float32)
    h = jnp.maximum(h, 0.0).astype(jnp.bfloat16)
    h = jnp.dot(h, w2_ref[...], preferred_element_type=jnp.float32) + b2_ref[...]
    h = jnp.maximum(h, 0.0).astype(jnp.bfloat16)
    p = jnp.dot(h, w3_ref[...], preferred_element_type=jnp.float32) + b3_ref[...]
    p = p.reshape(four, tm, p.shape[1])
    e = r_ref[:, :, 4:5].astype(jnp.float32)
    o_ref[...] = jnp.sum(p * e, axis=0).astype(o_ref.dtype)


def kernel(w_conv, b_conv, w1, b1, w2, b2, w3, b3, inp, coord, cell):
    B, Cin, H, W = inp.shape
    Cf = w_conv.shape[0]
    D, Hd = w1.shape
    Od = w3.shape[1]
    Q = coord.shape[1]
    BQ = B * Q
    C9 = Cf * 9

    xp = jnp.pad(inp, ((0, 0), (0, 0), (1, 1), (1, 1)))
    patches = jnp.stack([xp[:, :, kh:kh + H, kw:kw + W]
                         for kh in range(3) for kw in range(3)], axis=2)
    cols = (patches.reshape(B, Cin * 9, H, W).transpose(0, 2, 3, 1)
            .reshape(B, H * W, Cin * 9).astype(jnp.bfloat16))
    wce = w_conv.reshape(Cf, Cin * 9).T.astype(jnp.bfloat16)
    bce = b_conv.reshape(1, Cf).astype(jnp.float32)
    w1f = (w1[:C9].reshape(Cf, 9, Hd).transpose(1, 0, 2)
           .reshape(C9, Hd).astype(jnp.bfloat16))
    b1r = b1.reshape(1, Hd).astype(jnp.float32)

    th = 16 if H % 16 == 0 else H
    c1 = pl.pallas_call(
        _enc_conv1_kernel,
        out_shape=jax.ShapeDtypeStruct((B, H * W, Hd), jnp.bfloat16),
        grid=(B, H // th),
        in_specs=[
            pl.BlockSpec((1, H * W, Cin * 9), lambda b, t: (b, 0, 0)),
            pl.BlockSpec((Cin * 9, Cf), lambda b, t: (0, 0)),
            pl.BlockSpec((1, Cf), lambda b, t: (0, 0)),
            pl.BlockSpec((C9, Hd), lambda b, t: (0, 0)),
            pl.BlockSpec((1, Hd), lambda b, t: (0, 0)),
        ],
        out_specs=pl.BlockSpec((1, th * W, Hd), lambda b, t: (b, t, 0)),
        scratch_shapes=[pltpu.VMEM((H + 2, W + 2, Cf), jnp.bfloat16)],
        compiler_params=pltpu.CompilerParams(
            dimension_semantics=("parallel", "arbitrary")),
    )(cols, wce, bce, w1f, b1r)

    rx = 2.0 / H / 2.0
    ry = 2.0 / W / 2.0
    eps = 1e-6
    b_idx = jnp.arange(B, dtype=jnp.int32)[:, None]
    ixs, iys, rels, areas = [], [], [], []
    for vx in (-1, 1):
        for vy in (-1, 1):
            cs = coord.at[..., 0].add(vx * rx + eps)
            cs = cs.at[..., 1].add(vy * ry + eps)
            cs = jnp.clip(cs, -1 + 1e-6, 1 - 1e-6)
            ix = jnp.clip(jnp.floor((cs[..., 0] + 1.0) * 0.5 * H),
                          0, H - 1).astype(jnp.int32)
            iy = jnp.clip(jnp.floor((cs[..., 1] + 1.0) * 0.5 * W),
                          0, W - 1).astype(jnp.int32)
            qc_x = -1.0 + (2.0 * ix.astype(jnp.float32) + 1.0) / H
            qc_y = -1.0 + (2.0 * iy.astype(jnp.float32) + 1.0) / W
            rel_x = (coord[..., 0] - qc_x) * H
            rel_y = (coord[..., 1] - qc_y) * W
            ixs.append(ix)
            iys.append(iy)
            rels.append((rel_x, rel_y))
            areas.append(jnp.abs(rel_x * rel_y) + 1e-9)
    tot = areas[0] + areas[1] + areas[2] + areas[3]
    rel_cell = cell * jnp.array([H, W], dtype=jnp.float32)

    cy = jnp.minimum(iys[0], W - 2)
    starts = jnp.stack([
        (b_idx * (H * W) + ixs[0] * W + cy).reshape(BQ),
        (b_idx * (H * W) + ixs[2] * W + cy).reshape(BQ)], 0)
    ocs = [(iy - cy).astype(jnp.float32) for iy in iys]

    z = jnp.zeros_like(tot)
    rfeat = jnp.stack([
        jnp.stack([rels[s][0], rels[s][1], rel_cell[..., 0], rel_cell[..., 1],
                   areas[3 - s] / tot, ocs[s], z, z], axis=-1).reshape(BQ, 8)
        for s in range(4)], 0).astype(jnp.bfloat16)

    TM = 512 if BQ >= 512 else _ceil_to(BQ, 8)
    G = 1
    NC = BQ

    w1r = jnp.pad(w1[C9:], ((0, 8 - (D - C9)), (0, 0))).astype(jnp.bfloat16)
    w2b = w2.astype(jnp.bfloat16)
    b2r = b2.reshape(1, Hd).astype(jnp.float32)
    Np = _ceil_to(Od, 128)
    w3p = jnp.pad(w3, ((0, 0), (0, Np - Od))).astype(jnp.bfloat16)
    b3p = jnp.pad(b3.astype(jnp.float32), ((0, Np - Od),)).reshape(1, Np)

    c1f = c1.reshape(B * H * W, Hd)
    gd = jax.lax.GatherDimensionNumbers(
        offset_dims=(1, 2), collapsed_slice_dims=(), start_index_map=(0,))
    outs = []
    for gi in range(G):
        lo = gi * NC
        gg = jax.lax.gather(
            c1f, starts[:, lo:lo + NC].reshape(2 * NC, 1), gd,
            slice_sizes=(2, Hd),
            mode=jax.lax.GatherScatterMode.PROMISE_IN_BOUNDS)
        gg = gg.reshape(2, NC, 2 * Hd)
        rr = rfeat[:, lo:lo + NC]
        NCp = _ceil_to(NC, TM)
        if NCp != NC:
            gg = jnp.pad(gg, ((0, 0), (0, NCp - NC), (0, 0)))
            rr = jnp.pad(rr, ((0, 0), (0, NCp - NC), (0, 0)))
        o = pl.pallas_call(
            _query_mlp_kernel,
            out_shape=jax.ShapeDtypeStruct((NCp, Np), jnp.bfloat16),
            grid=(NCp // TM,),
            in_specs=[
                pl.BlockSpec((2, TM, 2 * Hd), lambda i: (0, i, 0)),
                pl.BlockSpec((4, TM, 8), lambda i: (0, i, 0)),
                pl.BlockSpec((8, Hd), lambda i: (0, 0)),
                pl.BlockSpec((Hd, Hd), lambda i: (0, 0)),
                pl.BlockSpec((1, Hd), lambda i: (0, 0)),
                pl.BlockSpec((Hd, Np), lambda i: (0, 0)),
                pl.BlockSpec((1, Np), lambda i: (0, 0)),
            ],
            out_specs=pl.BlockSpec((TM, Np), lambda i: (i, 0)),
            compiler_params=pltpu.CompilerParams(
                dimension_semantics=("parallel",)),
        )(gg, rr, w1r, w2b, b2r, w3p, b3p)
        outs.append(o[:NC, :Od])

    out = jnp.concatenate(outs, axis=0) if G > 1 else outs[0]
    return out[:BQ].astype(jnp.float32).reshape(B, Q, Od)


# speedup vs baseline: 57.0682x; 57.0682x over previous
import jax
import jax.numpy as jnp
from jax.experimental import pallas as pl
from jax.experimental.pallas import tpu as pltpu


def _ceil_to(x, m):
    return ((x + m - 1) // m) * m


def _enc_conv1_kernel(cols_ref, wce_ref, bce_ref, w1f_ref, b1_ref, o_ref,
                      feat_s):
    t = pl.program_id(1)
    hp2, w, c3 = feat_s.shape
    h = hp2 - 2
    cf = c3 // 3
    hd = o_ref.shape[2]
    th = o_ref.shape[1] // w

    @pl.when(t == 0)
    def _():
        f = jnp.dot(cols_ref[0], wce_ref[...],
                    preferred_element_type=jnp.float32) + bce_ref[...]
        v = f.reshape(h, w, cf).astype(feat_s.dtype)
        vp = jnp.pad(v, ((1, 1), (1, 1), (0, 0)))
        feat_s[...] = jnp.concatenate(
            [vp[:, 0:w, :], vp[:, 1:w + 1, :], vp[:, 2:w + 2, :]], axis=-1)

    r0 = t * th
    acc = jnp.zeros((th * w, hd), jnp.float32) + b1_ref[...]
    for kh in range(3):
        sl = feat_s[pl.ds(r0 + kh, th)]
        acc = acc + jnp.dot(sl.reshape(th * w, c3),
                            w1f_ref[kh * c3:(kh + 1) * c3, :],
                            preferred_element_type=jnp.float32)
    o_ref[0] = acc.astype(o_ref.dtype)


def _query_mlp_kernel(x_ref, r_ref, w1r_ref, w2_ref, b2_ref, w3_ref, b3_ref,
                      o_ref):
    four, tm, hd = x_ref.shape
    x = x_ref[...].reshape(four * tm, hd)
    rr = r_ref[...].reshape(four * tm, r_ref.shape[2])
    h = x.astype(jnp.float32) + jnp.dot(rr, w1r_ref[...],
                                        preferred_element_type=jnp.float32)
    h = jnp.maximum(h, 0.0).astype(jnp.bfloat16)
    h = jnp.dot(h, w2_ref[...], preferred_element_type=jnp.float32) + b2_ref[...]
    h = jnp.maximum(h, 0.0).astype(jnp.bfloat16)
    p = jnp.dot(h, w3_ref[...], preferred_element_type=jnp.float32) + b3_ref[...]
    p = p.reshape(four, tm, p.shape[1])
    e = r_ref[:, :, 4:5].astype(jnp.float32)
    o_ref[...] = jnp.sum(p * e, axis=0).astype(o_ref.dtype)


def kernel(w_conv, b_conv, w1, b1, w2, b2, w3, b3, inp, coord, cell):
    B, Cin, H, W = inp.shape
    Cf = w_conv.shape[0]
    D, Hd = w1.shape
    Od = w3.shape[1]
    Q = coord.shape[1]
    BQ = B * Q
    C9 = Cf * 9

    xp = jnp.pad(inp, ((0, 0), (0, 0), (1, 1), (1, 1)))
    patches = jnp.stack([xp[:, :, kh:kh + H, kw:kw + W]
                         for kh in range(3) for kw in range(3)], axis=2)
    cols = (patches.reshape(B, Cin * 9, H, W).transpose(0, 2, 3, 1)
            .reshape(B, H * W, Cin * 9).astype(jnp.bfloat16))
    wce = w_conv.reshape(Cf, Cin * 9).T.astype(jnp.bfloat16)
    bce = b_conv.reshape(1, Cf).astype(jnp.float32)
    w1f = (w1[:C9].reshape(Cf, 3, 3, Hd).transpose(1, 2, 0, 3)
           .reshape(C9, Hd).astype(jnp.bfloat16))
    b1r = b1.reshape(1, Hd).astype(jnp.float32)

    th = 16 if H % 16 == 0 else H
    c1 = pl.pallas_call(
        _enc_conv1_kernel,
        out_shape=jax.ShapeDtypeStruct((B, H * W, Hd), jnp.bfloat16),
        grid=(B, H // th),
        in_specs=[
            pl.BlockSpec((1, H * W, Cin * 9), lambda b, t: (b, 0, 0)),
            pl.BlockSpec((Cin * 9, Cf), lambda b, t: (0, 0)),
            pl.BlockSpec((1, Cf), lambda b, t: (0, 0)),
            pl.BlockSpec((C9, Hd), lambda b, t: (0, 0)),
            pl.BlockSpec((1, Hd), lambda b, t: (0, 0)),
        ],
        out_specs=pl.BlockSpec((1, th * W, Hd), lambda b, t: (b, t, 0)),
        scratch_shapes=[pltpu.VMEM((H + 2, W, 3 * Cf), jnp.bfloat16)],
        compiler_params=pltpu.CompilerParams(
            dimension_semantics=("parallel", "arbitrary"),
            vmem_limit_bytes=44 * 1024 * 1024),
    )(cols, wce, bce, w1f, b1r)

    rx = 2.0 / H / 2.0
    ry = 2.0 / W / 2.0
    eps = 1e-6
    b_idx = jnp.arange(B, dtype=jnp.int32)[:, None]
    ixs, iys, rels, areas = [], [], [], []
    for vx in (-1, 1):
        for vy in (-1, 1):
            cs = coord.at[..., 0].add(vx * rx + eps)
            cs = cs.at[..., 1].add(vy * ry + eps)
            cs = jnp.clip(cs, -1 + 1e-6, 1 - 1e-6)
            ix = jnp.clip(jnp.floor((cs[..., 0] + 1.0) * 0.5 * H),
                          0, H - 1).astype(jnp.int32)
            iy = jnp.clip(jnp.floor((cs[..., 1] + 1.0) * 0.5 * W),
                          0, W - 1).astype(jnp.int32)
            qc_x = -1.0 + (2.0 * ix.astype(jnp.float32) + 1.0) / H
            qc_y = -1.0 + (2.0 * iy.astype(jnp.float32) + 1.0) / W
            rel_x = (coord[..., 0] - qc_x) * H
            rel_y = (coord[..., 1] - qc_y) * W
            ixs.append(ix)
            iys.append(iy)
            rels.append((rel_x, rel_y))
            areas.append(jnp.abs(rel_x * rel_y) + 1e-9)
    tot = areas[0] + areas[1] + areas[2] + areas[3]
    rel_cell = cell * jnp.array([H, W], dtype=jnp.float32)

    idx_all = jnp.stack([
        (b_idx * (H * W) + ixs[s] * W + iys[s]).reshape(BQ)
        for s in range(4)], 0)

    z = jnp.zeros_like(tot)
    rfeat = jnp.stack([
        jnp.stack([rels[s][0], rels[s][1], rel_cell[..., 0], rel_cell[..., 1],
                   areas[3 - s] / tot, z, z, z], axis=-1).reshape(BQ, 8)
        for s in range(4)], 0).astype(jnp.bfloat16)

    TM = 512 if BQ >= 512 else _ceil_to(BQ, 8)
    G = 1
    NC = BQ

    w1r = jnp.pad(w1[C9:], ((0, 8 - (D - C9)), (0, 0))).astype(jnp.bfloat16)
    w2b = w2.astype(jnp.bfloat16)
    b2r = b2.reshape(1, Hd).astype(jnp.float32)
    Np = _ceil_to(Od, 128)
    w3p = jnp.pad(w3, ((0, 0), (0, Np - Od))).astype(jnp.bfloat16)
    b3p = jnp.pad(b3.astype(jnp.float32), ((0, Np - Od),)).reshape(1, Np)

    c1f = c1.reshape(B * H * W, Hd)
    outs = []
    for gi in range(G):
        lo = gi * NC
        gg = jnp.take(c1f, idx_all[:, lo:lo + NC].reshape(4 * NC), axis=0)
        gg = gg.reshape(4, NC, Hd)
        rr = rfeat[:, lo:lo + NC]
        NCp = _ceil_to(NC, TM)
        if NCp != NC:
            gg = jnp.pad(gg, ((0, 0), (0, NCp - NC), (0, 0)))
            rr = jnp.pad(rr, ((0, 0), (0, NCp - NC), (0, 0)))
        o = pl.pallas_call(
            _query_mlp_kernel,
            out_shape=jax.ShapeDtypeStruct((NCp, Np), jnp.bfloat16),
            grid=(NCp // TM,),
            in_specs=[
                pl.BlockSpec((4, TM, Hd), lambda i: (0, i, 0)),
                pl.BlockSpec((4, TM, 8), lambda i: (0, i, 0)),
                pl.BlockSpec((8, Hd), lambda i: (0, 0)),
                pl.BlockSpec((Hd, Hd), lambda i: (0, 0)),
                pl.BlockSpec((1, Hd), lambda i: (0, 0)),
                pl.BlockSpec((Hd, Np), lambda i: (0, 0)),
                pl.BlockSpec((1, Np), lambda i: (0, 0)),
            ],
            out_specs=pl.BlockSpec((TM, Np), lambda i: (i, 0)),
            compiler_params=pltpu.CompilerParams(
                dimension_semantics=("parallel",)),
        )(gg, rr, w1r, w2b, b2r, w3p, b3p)
        outs.append(o[:NC, :Od])

    out = jnp.concatenate(outs, axis=0) if G > 1 else outs[0]
    return out[:BQ].astype(jnp.float32).reshape(B, Q, Od)


# speedup vs baseline: 60.4564x; 1.0594x over previous
import jax
import jax.numpy as jnp
from jax.experimental import pallas as pl
from jax.experimental.pallas import tpu as pltpu


def _ceil_to(x, m):
    return ((x + m - 1) // m) * m


def _enc_conv1_kernel(cols_ref, wce_ref, bce_ref, w1f_ref, b1_ref, o_ref,
                      feat_s):
    t = pl.program_id(1)
    hp2, w, c3 = feat_s.shape
    h = hp2 - 2
    cf = c3 // 3
    hd = o_ref.shape[2]
    th = o_ref.shape[1] // w

    @pl.when(t == 0)
    def _():
        f = jnp.dot(cols_ref[0], wce_ref[...],
                    preferred_element_type=jnp.float32) + bce_ref[...]
        v = f.reshape(h, w, cf).astype(feat_s.dtype)
        vp = jnp.pad(v, ((1, 1), (1, 1), (0, 0)))
        feat_s[...] = jnp.concatenate(
            [vp[:, 0:w, :], vp[:, 1:w + 1, :], vp[:, 2:w + 2, :]], axis=-1)

    r0 = t * th
    acc = jnp.zeros((th * w, hd), jnp.float32) + b1_ref[...]
    for kh in range(3):
        sl = feat_s[pl.ds(r0 + kh, th)]
        acc = acc + jnp.dot(sl.reshape(th * w, c3),
                            w1f_ref[kh * c3:(kh + 1) * c3, :],
                            preferred_element_type=jnp.float32)
    o_ref[0] = acc.astype(o_ref.dtype)


def _query_mlp_kernel(x_ref, r_ref, w1r_ref, w2_ref, b2_ref, w3_ref, b3_ref,
                      o_ref):
    four, tm, hd = x_ref.shape
    x = x_ref[...].reshape(four * tm, hd)
    rr = r_ref[...].reshape(four * tm, r_ref.shape[2])
    h = x.astype(jnp.float32) + jnp.dot(rr, w1r_ref[...],
                                        preferred_element_type=jnp.float32)
    h = jnp.maximum(h, 0.0).astype(jnp.bfloat16)
    h = jnp.dot(h, w2_ref[...], preferred_element_type=jnp.float32) + b2_ref[...]
    h = jnp.maximum(h, 0.0).astype(jnp.bfloat16)
    p = jnp.dot(h, w3_ref[...], preferred_element_type=jnp.float32) + b3_ref[...]
    p = p.reshape(four, tm, p.shape[1])
    e = r_ref[:, :, 4:5].astype(jnp.float32)
    o_ref[...] = jnp.sum(p * e, axis=0).astype(o_ref.dtype)


def kernel(w_conv, b_conv, w1, b1, w2, b2, w3, b3, inp, coord, cell):
    B, Cin, H, W = inp.shape
    Cf = w_conv.shape[0]
    D, Hd = w1.shape
    Od = w3.shape[1]
    Q = coord.shape[1]
    BQ = B * Q
    C9 = Cf * 9

    xp = jnp.pad(inp, ((0, 0), (0, 0), (1, 1), (1, 1)))
    patches = jnp.stack([xp[:, :, kh:kh + H, kw:kw + W]
                         for kh in range(3) for kw in range(3)], axis=2)
    cols = (patches.reshape(B, Cin * 9, H, W).transpose(0, 2, 3, 1)
            .reshape(B, H * W, Cin * 9).astype(jnp.bfloat16))
    wce = w_conv.reshape(Cf, Cin * 9).T.astype(jnp.bfloat16)
    bce = b_conv.reshape(1, Cf).astype(jnp.float32)
    w1f = (w1[:C9].reshape(Cf, 3, 3, Hd).transpose(1, 2, 0, 3)
           .reshape(C9, Hd).astype(jnp.bfloat16))
    b1r = b1.reshape(1, Hd).astype(jnp.float32)

    th = 16 if H % 16 == 0 else H
    c1 = pl.pallas_call(
        _enc_conv1_kernel,
        out_shape=jax.ShapeDtypeStruct((B, H * W, Hd), jnp.bfloat16),
        grid=(B, H // th),
        in_specs=[
            pl.BlockSpec((1, H * W, Cin * 9), lambda b, t: (b, 0, 0)),
            pl.BlockSpec((Cin * 9, Cf), lambda b, t: (0, 0)),
            pl.BlockSpec((1, Cf), lambda b, t: (0, 0)),
            pl.BlockSpec((C9, Hd), lambda b, t: (0, 0)),
            pl.BlockSpec((1, Hd), lambda b, t: (0, 0)),
        ],
        out_specs=pl.BlockSpec((1, th * W, Hd), lambda b, t: (b, t, 0)),
        scratch_shapes=[pltpu.VMEM((H + 2, W, 3 * Cf), jnp.bfloat16)],
        compiler_params=pltpu.CompilerParams(
            dimension_semantics=("parallel", "arbitrary"),
            vmem_limit_bytes=44 * 1024 * 1024),
    )(cols, wce, bce, w1f, b1r)

    rx = 2.0 / H / 2.0
    ry = 2.0 / W / 2.0
    eps = 1e-6
    b_idx = jnp.arange(B, dtype=jnp.int32)[:, None]
    ixs, iys, rels, areas = [], [], [], []
    for vx in (-1, 1):
        for vy in (-1, 1):
            cs = coord.at[..., 0].add(vx * rx + eps)
            cs = cs.at[..., 1].add(vy * ry + eps)
            cs = jnp.clip(cs, -1 + 1e-6, 1 - 1e-6)
            ix = jnp.clip(jnp.floor((cs[..., 0] + 1.0) * 0.5 * H),
                          0, H - 1).astype(jnp.int32)
            iy = jnp.clip(jnp.floor((cs[..., 1] + 1.0) * 0.5 * W),
                          0, W - 1).astype(jnp.int32)
            qc_x = -1.0 + (2.0 * ix.astype(jnp.float32) + 1.0) / H
            qc_y = -1.0 + (2.0 * iy.astype(jnp.float32) + 1.0) / W
            rel_x = (coord[..., 0] - qc_x) * H
            rel_y = (coord[..., 1] - qc_y) * W
            ixs.append(ix)
            iys.append(iy)
            rels.append((rel_x, rel_y))
            areas.append(jnp.abs(rel_x * rel_y) + 1e-9)
    tot = areas[0] + areas[1] + areas[2] + areas[3]
    rel_cell = cell * jnp.array([H, W], dtype=jnp.float32)

    idx_all = jnp.stack([
        (b_idx * (H * W) + ixs[s] * W + iys[s]).reshape(BQ)
        for s in range(4)], 0)

    z = jnp.zeros_like(tot)
    rfeat = jnp.stack([
        jnp.stack([rels[s][0], rels[s][1], rel_cell[..., 0], rel_cell[..., 1],
                   areas[3 - s] / tot, z, z, z], axis=-1).reshape(BQ, 8)
        for s in range(4)], 0).astype(jnp.bfloat16)

    TM = 1024 if BQ >= 1024 else _ceil_to(BQ, 8)
    G = 1
    NC = BQ

    w1r = jnp.pad(w1[C9:], ((0, 8 - (D - C9)), (0, 0))).astype(jnp.bfloat16)
    w2b = w2.astype(jnp.bfloat16)
    b2r = b2.reshape(1, Hd).astype(jnp.float32)
    Np = _ceil_to(Od, 128)
    w3p = jnp.pad(w3, ((0, 0), (0, Np - Od))).astype(jnp.bfloat16)
    b3p = jnp.pad(b3.astype(jnp.float32), ((0, Np - Od),)).reshape(1, Np)

    c1f = c1.reshape(B * H * W, Hd)
    outs = []
    for gi in range(G):
        lo = gi * NC
        gg = jnp.take(c1f, idx_all[:, lo:lo + NC].reshape(4 * NC), axis=0)
        gg = gg.reshape(4, NC, Hd)
        rr = rfeat[:, lo:lo + NC]
        NCp = _ceil_to(NC, TM)
        if NCp != NC:
            gg = jnp.pad(gg, ((0, 0), (0, NCp - NC), (0, 0)))
            rr = jnp.pad(rr, ((0, 0), (0, NCp - NC), (0, 0)))
        o = pl.pallas_call(
            _query_mlp_kernel,
            out_shape=jax.ShapeDtypeStruct((NCp, Np), jnp.bfloat16),
            grid=(NCp // TM,),
            in_specs=[
                pl.BlockSpec((4, TM, Hd), lambda i: (0, i, 0)),
                pl.BlockSpec((4, TM, 8), lambda i: (0, i, 0)),
                pl.BlockSpec((8, Hd), lambda i: (0, 0)),
                pl.BlockSpec((Hd, Hd), lambda i: (0, 0)),
                pl.BlockSpec((1, Hd), lambda i: (0, 0)),
                pl.BlockSpec((Hd, Np), lambda i: (0, 0)),
                pl.BlockSpec((1, Np), lambda i: (0, 0)),
            ],
            out_specs=pl.BlockSpec((TM, Np), lambda i: (i, 0)),
            compiler_params=pltpu.CompilerParams(
                dimension_semantics=("parallel",)),
        )(gg, rr, w1r, w2b, b2r, w3p, b3p)
        outs.append(o[:NC, :Od])

    out = jnp.concatenate(outs, axis=0) if G > 1 else outs[0]
    return out[:BQ].astype(jnp.float32).reshape(B, Q, Od)


# speedup vs baseline: 61.6865x; 1.0203x over previous
import jax
import jax.numpy as jnp
from jax.experimental import pallas as pl
from jax.experimental.pallas import tpu as pltpu


def _ceil_to(x, m):
    return ((x + m - 1) // m) * m


def _enc_conv1_kernel(cols_ref, wce_ref, bce_ref, w1f_ref, b1_ref, o_ref,
                      feat_s):
    t = pl.program_id(1)
    hp2, w, c3 = feat_s.shape
    h = hp2 - 2
    cf = c3 // 3
    hd = o_ref.shape[2]
    th = o_ref.shape[1] // w

    @pl.when(t == 0)
    def _():
        f = jnp.dot(cols_ref[0], wce_ref[...],
                    preferred_element_type=jnp.float32) + bce_ref[...]
        v = f.reshape(h, w, cf).astype(feat_s.dtype)
        vp = jnp.pad(v, ((1, 1), (1, 1), (0, 0)))
        feat_s[...] = jnp.concatenate(
            [vp[:, 0:w, :], vp[:, 1:w + 1, :], vp[:, 2:w + 2, :]], axis=-1)

    r0 = t * th
    acc = jnp.zeros((th * w, hd), jnp.float32) + b1_ref[...]
    for kh in range(3):
        sl = feat_s[pl.ds(r0 + kh, th)]
        acc = acc + jnp.dot(sl.reshape(th * w, c3),
                            w1f_ref[kh * c3:(kh + 1) * c3, :],
                            preferred_element_type=jnp.float32)
    o_ref[0] = acc.astype(o_ref.dtype)


def _query_mlp_kernel(x_ref, r_ref, w1r_ref, w2_ref, b2_ref, w3_ref, b3_ref,
                      o_ref):
    four, tm, hd = x_ref.shape
    x = x_ref[...].reshape(four * tm, hd)
    rr = r_ref[...].reshape(four * tm, r_ref.shape[2])
    h = x.astype(jnp.float32) + jnp.dot(rr, w1r_ref[...],
                                        preferred_element_type=jnp.float32)
    h = jnp.maximum(h, 0.0).astype(jnp.bfloat16)
    h = jnp.dot(h, w2_ref[...], preferred_element_type=jnp.float32) + b2_ref[...]
    h = jnp.maximum(h, 0.0).astype(jnp.bfloat16)
    p = jnp.dot(h, w3_ref[...], preferred_element_type=jnp.float32) + b3_ref[...]
    p = p.reshape(four, tm, p.shape[1])
    e = r_ref[:, :, 4:5].astype(jnp.float32)
    o_ref[...] = jnp.sum(p * e, axis=0).astype(o_ref.dtype)


def kernel(w_conv, b_conv, w1, b1, w2, b2, w3, b3, inp, coord, cell):
    B, Cin, H, W = inp.shape
    Cf = w_conv.shape[0]
    D, Hd = w1.shape
    Od = w3.shape[1]
    Q = coord.shape[1]
    BQ = B * Q
    C9 = Cf * 9

    xp = jnp.pad(inp, ((0, 0), (0, 0), (1, 1), (1, 1)))
    patches = jnp.stack([xp[:, :, kh:kh + H, kw:kw + W]
                         for kh in range(3) for kw in range(3)], axis=2)
    cols = (patches.reshape(B, Cin * 9, H, W).transpose(0, 2, 3, 1)
            .reshape(B, H * W, Cin * 9).astype(jnp.bfloat16))
    wce = w_conv.reshape(Cf, Cin * 9).T.astype(jnp.bfloat16)
    bce = b_conv.reshape(1, Cf).astype(jnp.float32)
    w1f = (w1[:C9].reshape(Cf, 3, 3, Hd).transpose(1, 2, 0, 3)
           .reshape(C9, Hd).astype(jnp.bfloat16))
    b1r = b1.reshape(1, Hd).astype(jnp.float32)

    th = 16 if H % 16 == 0 else H
    c1 = pl.pallas_call(
        _enc_conv1_kernel,
        out_shape=jax.ShapeDtypeStruct((B, H * W, Hd), jnp.bfloat16),
        grid=(B, H // th),
        in_specs=[
            pl.BlockSpec((1, H * W, Cin * 9), lambda b, t: (b, 0, 0)),
            pl.BlockSpec((Cin * 9, Cf), lambda b, t: (0, 0)),
            pl.BlockSpec((1, Cf), lambda b, t: (0, 0)),
            pl.BlockSpec((C9, Hd), lambda b, t: (0, 0)),
            pl.BlockSpec((1, Hd), lambda b, t: (0, 0)),
        ],
        out_specs=pl.BlockSpec((1, th * W, Hd), lambda b, t: (b, t, 0)),
        scratch_shapes=[pltpu.VMEM((H + 2, W, 3 * Cf), jnp.bfloat16)],
        compiler_params=pltpu.CompilerParams(
            dimension_semantics=("parallel", "arbitrary"),
            vmem_limit_bytes=44 * 1024 * 1024),
    )(cols, wce, bce, w1f, b1r)

    rx = 2.0 / H / 2.0
    ry = 2.0 / W / 2.0
    eps = 1e-6
    b_idx = jnp.arange(B, dtype=jnp.int32)[:, None]
    ixs, iys, rels, areas = [], [], [], []
    for vx in (-1, 1):
        for vy in (-1, 1):
            cs = coord.at[..., 0].add(vx * rx + eps)
            cs = cs.at[..., 1].add(vy * ry + eps)
            cs = jnp.clip(cs, -1 + 1e-6, 1 - 1e-6)
            ix = jnp.clip(jnp.floor((cs[..., 0] + 1.0) * 0.5 * H),
                          0, H - 1).astype(jnp.int32)
            iy = jnp.clip(jnp.floor((cs[..., 1] + 1.0) * 0.5 * W),
                          0, W - 1).astype(jnp.int32)
            qc_x = -1.0 + (2.0 * ix.astype(jnp.float32) + 1.0) / H
            qc_y = -1.0 + (2.0 * iy.astype(jnp.float32) + 1.0) / W
            rel_x = (coord[..., 0] - qc_x) * H
            rel_y = (coord[..., 1] - qc_y) * W
            ixs.append(ix)
            iys.append(iy)
            rels.append((rel_x, rel_y))
            areas.append(jnp.abs(rel_x * rel_y) + 1e-9)
    tot = areas[0] + areas[1] + areas[2] + areas[3]
    rel_cell = cell * jnp.array([H, W], dtype=jnp.float32)

    idx_all = jnp.stack([
        (b_idx * (H * W) + ixs[s] * W + iys[s]).reshape(BQ)
        for s in range(4)], 0)

    z = jnp.zeros_like(tot)
    rfeat = jnp.stack([
        jnp.stack([rels[s][0], rels[s][1], rel_cell[..., 0], rel_cell[..., 1],
                   areas[3 - s] / tot, z, z, z], axis=-1).reshape(BQ, 8)
        for s in range(4)], 0).astype(jnp.bfloat16)

    TM = 2048 if BQ >= 2048 else _ceil_to(BQ, 8)
    G = 1
    NC = BQ

    w1r = jnp.pad(w1[C9:], ((0, 8 - (D - C9)), (0, 0))).astype(jnp.bfloat16)
    w2b = w2.astype(jnp.bfloat16)
    b2r = b2.reshape(1, Hd).astype(jnp.float32)
    Np = _ceil_to(Od, 128)
    w3p = jnp.pad(w3, ((0, 0), (0, Np - Od))).astype(jnp.bfloat16)
    b3p = jnp.pad(b3.astype(jnp.float32), ((0, Np - Od),)).reshape(1, Np)

    c1f = c1.reshape(B * H * W, Hd)
    outs = []
    for gi in range(G):
        lo = gi * NC
        gg = jnp.take(c1f, idx_all[:, lo:lo + NC].reshape(4 * NC), axis=0)
        gg = gg.reshape(4, NC, Hd)
        rr = rfeat[:, lo:lo + NC]
        NCp = _ceil_to(NC, TM)
        if NCp != NC:
            gg = jnp.pad(gg, ((0, 0), (0, NCp - NC), (0, 0)))
            rr = jnp.pad(rr, ((0, 0), (0, NCp - NC), (0, 0)))
        o = pl.pallas_call(
            _query_mlp_kernel,
            out_shape=jax.ShapeDtypeStruct((NCp, Np), jnp.bfloat16),
            grid=(NCp // TM,),
            in_specs=[
                pl.BlockSpec((4, TM, Hd), lambda i: (0, i, 0)),
                pl.BlockSpec((4, TM, 8), lambda i: (0, i, 0)),
                pl.BlockSpec((8, Hd), lambda i: (0, 0)),
                pl.BlockSpec((Hd, Hd), lambda i: (0, 0)),
                pl.BlockSpec((1, Hd), lambda i: (0, 0)),
                pl.BlockSpec((Hd, Np), lambda i: (0, 0)),
                pl.BlockSpec((1, Np), lambda i: (0, 0)),
            ],
            out_specs=pl.BlockSpec((TM, Np), lambda i: (i, 0)),
            compiler_params=pltpu.CompilerParams(
                dimension_semantics=("parallel",)),
        )(gg, rr, w1r, w2b, b2r, w3p, b3p)
        outs.append(o[:NC, :Od])

    out = jnp.concatenate(outs, axis=0) if G > 1 else outs[0]
    return out[:BQ].astype(jnp.float32).reshape(B, Q, Od)


# speedup vs baseline: 62.2151x; 1.0086x over previous
import jax
import jax.numpy as jnp
from jax.experimental import pallas as pl
from jax.experimental.pallas import tpu as pltpu


def _ceil_to(x, m):
    return ((x + m - 1) // m) * m


def _enc_conv1_kernel(cols_ref, wce_ref, bce_ref, w1f_ref, b1_ref, o_ref,
                      feat_s):
    t = pl.program_id(1)
    hp2, w, c3 = feat_s.shape
    h = hp2 - 2
    cf = c3 // 3
    hd = o_ref.shape[2]
    th = o_ref.shape[1] // w

    @pl.when(t == 0)
    def _():
        f = jnp.dot(cols_ref[0], wce_ref[...],
                    preferred_element_type=jnp.float32) + bce_ref[...]
        v = f.reshape(h, w, cf).astype(feat_s.dtype)
        vp = jnp.pad(v, ((1, 1), (1, 1), (0, 0)))
        feat_s[...] = jnp.concatenate(
            [vp[:, 0:w, :], vp[:, 1:w + 1, :], vp[:, 2:w + 2, :]], axis=-1)

    r0 = t * th
    acc = jnp.zeros((th * w, hd), jnp.float32) + b1_ref[...]
    for kh in range(3):
        sl = feat_s[pl.ds(r0 + kh, th)]
        acc = acc + jnp.dot(sl.reshape(th * w, c3),
                            w1f_ref[kh * c3:(kh + 1) * c3, :],
                            preferred_element_type=jnp.float32)
    o_ref[0] = acc.astype(o_ref.dtype)


def _query_mlp_kernel(x_ref, r_ref, w1r_ref, w2_ref, b2_ref, w3_ref, b3_ref,
                      o_ref):
    four, tm, hd = x_ref.shape
    x = x_ref[...].reshape(four * tm, hd)
    rr = r_ref[...].reshape(four * tm, r_ref.shape[2])
    h = x.astype(jnp.float32) + jnp.dot(rr, w1r_ref[...],
                                        preferred_element_type=jnp.float32)
    h = jnp.maximum(h, 0.0).astype(jnp.bfloat16)
    h = jnp.dot(h, w2_ref[...], preferred_element_type=jnp.float32) + b2_ref[...]
    h = jnp.maximum(h, 0.0).astype(jnp.bfloat16)
    p = jnp.dot(h, w3_ref[...], preferred_element_type=jnp.float32) + b3_ref[...]
    p = p.reshape(four, tm, p.shape[1])
    e = r_ref[:, :, 4:5].astype(jnp.float32)
    o_ref[...] = jnp.sum(p * e, axis=0).astype(o_ref.dtype)


def kernel(w_conv, b_conv, w1, b1, w2, b2, w3, b3, inp, coord, cell):
    B, Cin, H, W = inp.shape
    Cf = w_conv.shape[0]
    D, Hd = w1.shape
    Od = w3.shape[1]
    Q = coord.shape[1]
    BQ = B * Q
    C9 = Cf * 9

    xp = jnp.pad(inp, ((0, 0), (0, 0), (1, 1), (1, 1)))
    patches = jnp.stack([xp[:, :, kh:kh + H, kw:kw + W]
                         for kh in range(3) for kw in range(3)], axis=2)
    cols = (patches.reshape(B, Cin * 9, H, W).transpose(0, 2, 3, 1)
            .reshape(B, H * W, Cin * 9).astype(jnp.bfloat16))
    wce = w_conv.reshape(Cf, Cin * 9).T.astype(jnp.bfloat16)
    bce = b_conv.reshape(1, Cf).astype(jnp.float32)
    w1f = (w1[:C9].reshape(Cf, 3, 3, Hd).transpose(1, 2, 0, 3)
           .reshape(C9, Hd).astype(jnp.bfloat16))
    b1r = b1.reshape(1, Hd).astype(jnp.float32)

    th = 16 if H % 16 == 0 else H
    c1 = pl.pallas_call(
        _enc_conv1_kernel,
        out_shape=jax.ShapeDtypeStruct((B, H * W, Hd), jnp.bfloat16),
        grid=(B, H // th),
        in_specs=[
            pl.BlockSpec((1, H * W, Cin * 9), lambda b, t: (b, 0, 0)),
            pl.BlockSpec((Cin * 9, Cf), lambda b, t: (0, 0)),
            pl.BlockSpec((1, Cf), lambda b, t: (0, 0)),
            pl.BlockSpec((C9, Hd), lambda b, t: (0, 0)),
            pl.BlockSpec((1, Hd), lambda b, t: (0, 0)),
        ],
        out_specs=pl.BlockSpec((1, th * W, Hd), lambda b, t: (b, t, 0)),
        scratch_shapes=[pltpu.VMEM((H + 2, W, 3 * Cf), jnp.bfloat16)],
        compiler_params=pltpu.CompilerParams(
            dimension_semantics=("parallel", "arbitrary"),
            vmem_limit_bytes=44 * 1024 * 1024),
    )(cols, wce, bce, w1f, b1r)

    rx = 2.0 / H / 2.0
    ry = 2.0 / W / 2.0
    eps = 1e-6
    b_idx = jnp.arange(B, dtype=jnp.int32)[:, None]
    ixs, iys, rels, areas = [], [], [], []
    for vx in (-1, 1):
        for vy in (-1, 1):
            cs = coord.at[..., 0].add(vx * rx + eps)
            cs = cs.at[..., 1].add(vy * ry + eps)
            cs = jnp.clip(cs, -1 + 1e-6, 1 - 1e-6)
            ix = jnp.clip(jnp.floor((cs[..., 0] + 1.0) * 0.5 * H),
                          0, H - 1).astype(jnp.int32)
            iy = jnp.clip(jnp.floor((cs[..., 1] + 1.0) * 0.5 * W),
                          0, W - 1).astype(jnp.int32)
            qc_x = -1.0 + (2.0 * ix.astype(jnp.float32) + 1.0) / H
            qc_y = -1.0 + (2.0 * iy.astype(jnp.float32) + 1.0) / W
            rel_x = (coord[..., 0] - qc_x) * H
            rel_y = (coord[..., 1] - qc_y) * W
            ixs.append(ix)
            iys.append(iy)
            rels.append((rel_x, rel_y))
            areas.append(jnp.abs(rel_x * rel_y) + 1e-9)
    tot = areas[0] + areas[1] + areas[2] + areas[3]
    rel_cell = cell * jnp.array([H, W], dtype=jnp.float32)

    idx_all = jnp.stack([
        (b_idx * (H * W) + ixs[s] * W + iys[s]).reshape(BQ)
        for s in range(4)], 0)

    z = jnp.zeros_like(tot)
    rfeat = jnp.stack([
        jnp.stack([rels[s][0], rels[s][1], rel_cell[..., 0], rel_cell[..., 1],
                   areas[3 - s] / tot, z, z, z], axis=-1).reshape(BQ, 8)
        for s in range(4)], 0).astype(jnp.bfloat16)

    TM = 4096 if BQ >= 4096 else _ceil_to(BQ, 8)
    G = 1
    NC = BQ

    w1r = jnp.pad(w1[C9:], ((0, 8 - (D - C9)), (0, 0))).astype(jnp.bfloat16)
    w2b = w2.astype(jnp.bfloat16)
    b2r = b2.reshape(1, Hd).astype(jnp.float32)
    Np = _ceil_to(Od, 128)
    w3p = jnp.pad(w3, ((0, 0), (0, Np - Od))).astype(jnp.bfloat16)
    b3p = jnp.pad(b3.astype(jnp.float32), ((0, Np - Od),)).reshape(1, Np)

    c1f = c1.reshape(B * H * W, Hd)
    outs = []
    for gi in range(G):
        lo = gi * NC
        gg = jnp.take(c1f, idx_all[:, lo:lo + NC].reshape(4 * NC), axis=0)
        gg = gg.reshape(4, NC, Hd)
        rr = rfeat[:, lo:lo + NC]
        NCp = _ceil_to(NC, TM)
        if NCp != NC:
            gg = jnp.pad(gg, ((0, 0), (0, NCp - NC), (0, 0)))
            rr = jnp.pad(rr, ((0, 0), (0, NCp - NC), (0, 0)))
        o = pl.pallas_call(
            _query_mlp_kernel,
            out_shape=jax.ShapeDtypeStruct((NCp, Np), jnp.bfloat16),
            grid=(NCp // TM,),
            in_specs=[
                pl.BlockSpec((4, TM, Hd), lambda i: (0, i, 0)),
                pl.BlockSpec((4, TM, 8), lambda i: (0, i, 0)),
                pl.BlockSpec((8, Hd), lambda i: (0, 0)),
                pl.BlockSpec((Hd, Hd), lambda i: (0, 0)),
                pl.BlockSpec((1, Hd), lambda i: (0, 0)),
                pl.BlockSpec((Hd, Np), lambda i: (0, 0)),
                pl.BlockSpec((1, Np), lambda i: (0, 0)),
            ],
            out_specs=pl.BlockSpec((TM, Np), lambda i: (i, 0)),
            compiler_params=pltpu.CompilerParams(
                dimension_semantics=("parallel",)),
        )(gg, rr, w1r, w2b, b2r, w3p, b3p)
        outs.append(o[:NC, :Od])

    out = jnp.concatenate(outs, axis=0) if G > 1 else outs[0]
    return out[:BQ].astype(jnp.float32).reshape(B, Q, Od)
